# Initial kernel scaffold; baseline (speedup 1.0000x reference)
import jax
import jax.numpy as jnp
from jax.experimental import pallas as pl


def kernel(image):
    raise NotImplementedError("write your pallas kernel here")


# speedup vs baseline: 12.3395x; 12.3395x over previous
import functools

import jax
import jax.numpy as jnp
from jax import lax
from jax.experimental import pallas as pl
from jax.experimental.pallas import tpu as pltpu

_B, _C, _H, _W = 8, 3, 384, 384
_N = _H * _W
_K = 1327
_PAD = 3
_CLIP = 0.89


def _dark_channel(x):
    cmin = jnp.min(x, axis=0)
    inf_rows = jnp.full((_PAD, _W), jnp.inf, dtype=cmin.dtype)
    padv = jnp.concatenate([inf_rows, cmin, inf_rows], axis=0)
    vmin = padv[0:_H]
    for dy in range(1, 2 * _PAD + 1):
        vmin = jnp.minimum(vmin, padv[dy:dy + _H])
    inf_cols = jnp.full((_H, _PAD), jnp.inf, dtype=cmin.dtype)
    padh = jnp.concatenate([inf_cols, vmin, inf_cols], axis=1)
    hmin = padh[:, 0:_W]
    for dx in range(1, 2 * _PAD + 1):
        hmin = jnp.minimum(hmin, padh[:, dx:dx + _W])
    return hmin


def _airlight_kernel(img_ref, out_ref):
    b = pl.program_id(0)
    x = img_ref[0]
    dc = _dark_channel(x)
    dc_bits = lax.bitcast_convert_type(dc, jnp.int32)

    def val_body(_, carry):
        lo, hi = carry
        mid = lo + ((hi - lo) >> 1)
        cnt = jnp.sum((dc_bits >= mid).astype(jnp.int32))
        take = cnt >= _K
        return (jnp.where(take, mid, lo), jnp.where(take, hi, mid))

    lo0 = jnp.int32(0)
    hi0 = jnp.int32(0x7F800000)
    t_bits, _ = lax.fori_loop(0, 31, val_body, (lo0, hi0))

    cnt_gt = jnp.sum((dc_bits > t_bits).astype(jnp.int32))
    m = _K - cnt_gt

    eq = dc_bits == t_bits
    idx = (lax.broadcasted_iota(jnp.int32, (_H, _W), 0) * _W
           + lax.broadcasted_iota(jnp.int32, (_H, _W), 1))

    def idx_body(_, carry):
        lo2, hi2 = carry
        mid = lo2 + ((hi2 - lo2) >> 1)
        c = jnp.sum((eq & (idx <= mid)).astype(jnp.int32))
        ge = c >= m
        return (jnp.where(ge, lo2, mid), jnp.where(ge, mid, hi2))

    _, j_star = lax.fori_loop(0, 18, idx_body, (jnp.int32(-1), jnp.int32(_N - 1)))
    sel = (dc_bits > t_bits) | (eq & (idx <= j_star))

    masked = jnp.where(sel[None, :, :], x, -jnp.inf)
    airlight = jnp.minimum(jnp.max(masked, axis=(1, 2)), _CLIP)
    contrib = jnp.sum(airlight) / (_B * _C)

    @pl.when(b == 0)
    def _():
        out_ref[...] = jnp.zeros((1, 1), jnp.float32)

    out_ref[...] += contrib[None, None]


@jax.jit
def kernel(image):
    out = pl.pallas_call(
        _airlight_kernel,
        grid=(_B,),
        in_specs=[pl.BlockSpec((1, _C, _H, _W), lambda b: (b, 0, 0, 0))],
        out_specs=pl.BlockSpec((1, 1), lambda b: (0, 0)),
        out_shape=jax.ShapeDtypeStruct((1, 1), jnp.float32),
        compiler_params=pltpu.CompilerParams(
            dimension_semantics=("arbitrary",),
        ),
    )(image)
    return out.reshape(())
